# baseline (speedup 1.0000x reference)
import functools

import jax
import jax.numpy as jnp
from jax.experimental import pallas as pl
from jax.experimental.pallas import tpu as pltpu

B = 8192
STATE_DIM = 768
NUM_OPTIONS = 16
NUM_ACTIONS = 8
H = 128
BLK = 256


def _first_argmax(x, width):
    idx = jax.lax.broadcasted_iota(jnp.int32, x.shape, len(x.shape) - 1)
    maxv = jnp.max(x, axis=-1, keepdims=True)
    cand = jnp.where(x == maxv, idx, width)
    return jnp.min(cand, axis=-1, keepdims=True)


def _dense_body(state_ref, W1s_ref, b1s_ref, W2s_ref, b2s_ref,
                Wopt1_ref, bopt1_ref, Wopt2_ref, bopt2_ref,
                Wt1_ref, bt1_ref, Wt2_ref, bt2_ref, sel_ref,
                opt_probs_ref, opt_arg_ref, act_probs_ref, act_arg_ref,
                term_ref):
    x = state_ref[...]

    h = jnp.maximum(
        jnp.dot(x, Wopt1_ref[...], preferred_element_type=jnp.float32)
        + bopt1_ref[...], 0.0)
    lg = (jnp.dot(h, Wopt2_ref[...], preferred_element_type=jnp.float32)
          + bopt2_ref[...])
    m = jnp.max(lg, axis=-1, keepdims=True)
    ex = jnp.exp(lg - m)
    opt_probs_ref[...] = ex / jnp.sum(ex, axis=-1, keepdims=True)
    opt_arg_ref[...] = _first_argmax(lg, NUM_OPTIONS)

    ht = jnp.maximum(
        jnp.dot(x, Wt1_ref[...], preferred_element_type=jnp.float32)
        + bt1_ref[...], 0.0)
    term_ref[...] = jax.nn.sigmoid(
        jnp.dot(ht, Wt2_ref[...], preferred_element_type=jnp.float32)
        + bt2_ref[...])

    sel = sel_ref[...]
    q = jnp.zeros((BLK, NUM_ACTIONS), jnp.float32)
    for e in range(NUM_OPTIONS):
        h1 = jnp.maximum(
            jnp.dot(x, W1s_ref[e], preferred_element_type=jnp.float32)
            + b1s_ref[e:e + 1, :], 0.0)
        qe = (jnp.dot(h1, W2s_ref[e], preferred_element_type=jnp.float32)
              + b2s_ref[e:e + 1, :])
        q = jnp.where(sel == e, qe, q)

    mq = jnp.max(q, axis=-1, keepdims=True)
    eq = jnp.exp(q - mq)
    act_probs_ref[...] = eq / jnp.sum(eq, axis=-1, keepdims=True)
    act_arg_ref[...] = _first_argmax(q, NUM_ACTIONS)


def kernel(state, Wopt1, bopt1, Wopt2, bopt2, W1s, b1s, W2s, b2s,
           Wt1, bt1, Wt2, bt2):
    h_x = jnp.maximum(state @ Wopt1 + bopt1, 0.0)
    logits_x = h_x @ Wopt2 + bopt2
    skey = jax.random.key(42)
    sel = jax.random.categorical(
        skey, jax.lax.stop_gradient(logits_x), axis=-1)

    nblk = B // BLK
    full = lambda i: (0, 0)
    full3 = lambda i: (0, 0, 0)
    out = pl.pallas_call(
        _dense_body,
        grid=(nblk,),
        in_specs=[
            pl.BlockSpec((BLK, STATE_DIM), lambda i: (i, 0)),
            pl.BlockSpec((NUM_OPTIONS, STATE_DIM, H), full3),
            pl.BlockSpec((NUM_OPTIONS, H), full),
            pl.BlockSpec((NUM_OPTIONS, H, NUM_ACTIONS), full3),
            pl.BlockSpec((NUM_OPTIONS, NUM_ACTIONS), full),
            pl.BlockSpec((STATE_DIM, H), full),
            pl.BlockSpec((1, H), full),
            pl.BlockSpec((H, NUM_OPTIONS), full),
            pl.BlockSpec((1, NUM_OPTIONS), full),
            pl.BlockSpec((STATE_DIM, H), full),
            pl.BlockSpec((1, H), full),
            pl.BlockSpec((H, 1), full),
            pl.BlockSpec((1, 1), full),
            pl.BlockSpec((BLK, 1), lambda i: (i, 0)),
        ],
        out_specs=[
            pl.BlockSpec((BLK, NUM_OPTIONS), lambda i: (i, 0)),
            pl.BlockSpec((BLK, 1), lambda i: (i, 0)),
            pl.BlockSpec((BLK, NUM_ACTIONS), lambda i: (i, 0)),
            pl.BlockSpec((BLK, 1), lambda i: (i, 0)),
            pl.BlockSpec((BLK, 1), lambda i: (i, 0)),
        ],
        out_shape=[
            jax.ShapeDtypeStruct((B, NUM_OPTIONS), jnp.float32),
            jax.ShapeDtypeStruct((B, 1), jnp.int32),
            jax.ShapeDtypeStruct((B, NUM_ACTIONS), jnp.float32),
            jax.ShapeDtypeStruct((B, 1), jnp.int32),
            jax.ShapeDtypeStruct((B, 1), jnp.float32),
        ],
    )(state, W1s, b1s, W2s, b2s,
      Wopt1, bopt1.reshape(1, H), Wopt2, bopt2.reshape(1, NUM_OPTIONS),
      Wt1, bt1.reshape(1, H), Wt2, bt2.reshape(1, 1),
      sel.astype(jnp.int32).reshape(B, 1))

    opt_probs, opt_arg, act_probs, act_arg, term = out
    return (opt_probs, act_probs, term,
            opt_arg.reshape(B), act_arg.reshape(B))
